# Initial kernel scaffold; baseline (speedup 1.0000x reference)
import jax
import jax.numpy as jnp
from jax.experimental import pallas as pl


def kernel(x, segment_ids, segment_table):
    raise NotImplementedError("write your pallas kernel here")


# speedup vs baseline: 3.0049x; 3.0049x over previous
import functools

import jax
import jax.numpy as jnp
from jax import lax
from jax.experimental import pallas as pl
from jax.experimental.pallas import tpu as pltpu
from jax.experimental.pallas import tpu_sc as plsc

_D = 64
_L = 16
_NSEG = 3
_NC, _NS = 2, 16
_NW = _NC * _NS
_C = 800


def _sc_body(x_hbm, ids_hbm, tab_hbm, out_hbm,
             xbuf, idsbuf, tabv, sem_in0, sem_in1, sem_out0, sem_out1):
    n = x_hbm.shape[0]
    tpw = n // _NW
    steps = tpw // _C
    npairs = steps // 2
    wid = lax.axis_index("s") * _NC + lax.axis_index("c")
    base = wid * tpw

    pltpu.sync_copy(tab_hbm, tabv)
    base_j = [lax.iota(jnp.int32, _L) + j * _L for j in range(_D // _L)]

    sems_in = (sem_in0, sem_in1)
    sems_out = (sem_out0, sem_out1)

    def start_in(g, slot):
        row0 = base + g * _C
        pltpu.async_copy(x_hbm.at[pl.ds(row0, _C)], xbuf.at[slot],
                         sems_in[slot])
        pltpu.async_copy(ids_hbm.at[pl.ds(row0, _C)], idsbuf.at[slot],
                         sems_in[slot])

    def wait_in(slot):
        pltpu.make_async_copy(x_hbm.at[pl.ds(0, _C)], xbuf.at[slot],
                              sems_in[slot]).wait()
        pltpu.make_async_copy(ids_hbm.at[pl.ds(0, _C)], idsbuf.at[slot],
                              sems_in[slot]).wait()

    def start_out(g, slot):
        row0 = base + g * _C
        pltpu.async_copy(xbuf.at[slot], out_hbm.at[pl.ds(row0, _C)],
                         sems_out[slot])

    def wait_out(slot):
        pltpu.make_async_copy(xbuf.at[slot], out_hbm.at[pl.ds(0, _C)],
                              sems_out[slot]).wait()

    def compute(slot):
        @plsc.parallel_loop(0, _C, step=_L)
        def _(t0):
            offs16 = idsbuf[slot, pl.ds(t0, _L)] * _D
            for i in range(_L):
                offv = jnp.full((_L,), offs16[i], dtype=jnp.int32)
                for j in range(_D // _L):
                    row = plsc.load_gather(tabv, [offv + base_j[j]])
                    sl = pl.ds(j * _L, _L)
                    xbuf[slot, t0 + i, sl] = xbuf[slot, t0 + i, sl] + row

    start_in(0, 0)
    start_in(1, 1)

    def pair_body(gg, carry):
        g0 = 2 * gg
        wait_in(0)
        compute(0)
        start_out(g0, 0)
        wait_in(1)
        compute(1)
        start_out(g0 + 1, 1)

        @pl.when(gg + 1 < npairs)
        def _():
            wait_out(0)
            start_in(g0 + 2, 0)
            wait_out(1)
            start_in(g0 + 3, 1)

        return carry

    lax.fori_loop(0, npairs, pair_body, 0)
    wait_out(0)
    wait_out(1)


def kernel(x, segment_ids, segment_table):
    b, s, d = x.shape
    n = b * s
    x2 = x.reshape(n, d)
    ids = segment_ids.reshape(n).astype(jnp.int32)
    fn = pl.kernel(
        _sc_body,
        out_type=jax.ShapeDtypeStruct((n, d), jnp.float32),
        mesh=plsc.VectorSubcoreMesh(core_axis_name="c", subcore_axis_name="s",
                                    num_cores=_NC, num_subcores=_NS),
        compiler_params=pltpu.CompilerParams(needs_layout_passes=False,
                                             use_tc_tiling_on_sc=False),
        scratch_types=[
            pltpu.VMEM((2, _C, _D), jnp.float32),
            pltpu.VMEM((2, _C), jnp.int32),
            pltpu.VMEM((_NSEG * _D,), jnp.float32),
            pltpu.SemaphoreType.DMA,
            pltpu.SemaphoreType.DMA,
            pltpu.SemaphoreType.DMA,
            pltpu.SemaphoreType.DMA,
        ],
    )
    out = fn(x2, ids, segment_table.reshape(-1))
    return out.reshape(b, s, d)


# speedup vs baseline: 3.0993x; 1.0314x over previous
import functools

import jax
import jax.numpy as jnp
from jax import lax
from jax.experimental import pallas as pl
from jax.experimental.pallas import tpu as pltpu
from jax.experimental.pallas import tpu_sc as plsc

_D = 64
_L = 16
_NSEG = 3
_NC, _NS = 2, 16
_NW = _NC * _NS
_C = 800


def _sc_body(x_hbm, ids_hbm, tab_hbm, out_hbm,
             xbuf, idsbuf, tabv, sem_in0, sem_in1, sem_out0, sem_out1):
    n = x_hbm.shape[0]
    tpw = n // _NW
    steps = tpw // _C
    npairs = steps // 2
    wid = lax.axis_index("s") * _NC + lax.axis_index("c")
    base = wid * tpw

    pltpu.sync_copy(tab_hbm, tabv)
    trows = [[tabv[pl.ds(s * _D + j * _L, _L)] for j in range(_D // _L)]
             for s in range(_NSEG)]

    sems_in = (sem_in0, sem_in1)
    sems_out = (sem_out0, sem_out1)

    def start_in(g, slot):
        row0 = base + g * _C
        pltpu.async_copy(x_hbm.at[pl.ds(row0, _C)], xbuf.at[slot],
                         sems_in[slot])
        pltpu.async_copy(ids_hbm.at[pl.ds(row0, _C)], idsbuf.at[slot],
                         sems_in[slot])

    def wait_in(slot):
        pltpu.make_async_copy(x_hbm.at[pl.ds(0, _C)], xbuf.at[slot],
                              sems_in[slot]).wait()
        pltpu.make_async_copy(ids_hbm.at[pl.ds(0, _C)], idsbuf.at[slot],
                              sems_in[slot]).wait()

    def start_out(g, slot):
        row0 = base + g * _C
        pltpu.async_copy(xbuf.at[slot], out_hbm.at[pl.ds(row0, _C)],
                         sems_out[slot])

    def wait_out(slot):
        pltpu.make_async_copy(xbuf.at[slot], out_hbm.at[pl.ds(0, _C)],
                              sems_out[slot]).wait()

    def compute(slot):
        @plsc.parallel_loop(0, _C, step=_L)
        def _(t0):
            t0v = jnp.full((_L,), t0, dtype=jnp.int32)
            for i in range(_L):
                idv = plsc.load_gather(idsbuf.at[slot], [t0v + i])
                m0 = idv == 0
                m1 = idv == 1
                for j in range(_D // _L):
                    row = jnp.where(m0, trows[0][j],
                                    jnp.where(m1, trows[1][j], trows[2][j]))
                    sl = pl.ds(j * _L, _L)
                    xbuf[slot, t0 + i, sl] = xbuf[slot, t0 + i, sl] + row

    start_in(0, 0)
    start_in(1, 1)

    def pair_body(gg, carry):
        g0 = 2 * gg
        wait_in(0)
        compute(0)
        start_out(g0, 0)
        wait_in(1)
        compute(1)
        start_out(g0 + 1, 1)

        @pl.when(gg + 1 < npairs)
        def _():
            wait_out(0)
            start_in(g0 + 2, 0)
            wait_out(1)
            start_in(g0 + 3, 1)

        return carry

    lax.fori_loop(0, npairs, pair_body, 0)
    wait_out(0)
    wait_out(1)


def kernel(x, segment_ids, segment_table):
    b, s, d = x.shape
    n = b * s
    x2 = x.reshape(n, d)
    ids = segment_ids.reshape(n).astype(jnp.int32)
    fn = pl.kernel(
        _sc_body,
        out_type=jax.ShapeDtypeStruct((n, d), jnp.float32),
        mesh=plsc.VectorSubcoreMesh(core_axis_name="c", subcore_axis_name="s",
                                    num_cores=_NC, num_subcores=_NS),
        compiler_params=pltpu.CompilerParams(needs_layout_passes=False,
                                             use_tc_tiling_on_sc=False),
        scratch_types=[
            pltpu.VMEM((2, _C, _D), jnp.float32),
            pltpu.VMEM((2, _C), jnp.int32),
            pltpu.VMEM((_NSEG * _D,), jnp.float32),
            pltpu.SemaphoreType.DMA,
            pltpu.SemaphoreType.DMA,
            pltpu.SemaphoreType.DMA,
            pltpu.SemaphoreType.DMA,
        ],
    )
    out = fn(x2, ids, segment_table.reshape(-1))
    return out.reshape(b, s, d)
